# Initial kernel scaffold; baseline (speedup 1.0000x reference)
import jax
import jax.numpy as jnp
from jax.experimental import pallas as pl


def kernel(input_seq, target_item, table, W, b):
    raise NotImplementedError("write your pallas kernel here")


# speedup vs baseline: 8.2480x; 8.2480x over previous
import functools

import jax
import jax.numpy as jnp
from jax import lax
from jax.experimental import pallas as pl
from jax.experimental.pallas import tpu as pltpu
from jax.experimental.pallas import tpu_sc as plsc

_NC = 2
_NS = 16
_NW = _NC * _NS


@functools.lru_cache(maxsize=None)
def _make_proj(V1, D, BR=2048):
    grid = (V1 + BR - 1) // BR

    def body(tab_ref, wt_ref, s_ref, bz_ref, q1_ref, p2_ref):
        q = lax.dot_general(
            tab_ref[...], wt_ref[...],
            (((1,), (0,)), ((), ())),
            preferred_element_type=jnp.float32,
        )
        q = q * s_ref[...] + bz_ref[...]
        q1_ref[...] = q[:, 0:1]
        p2_ref[...] = q[:, 1:2]

    return pl.pallas_call(
        body,
        grid=(grid,),
        in_specs=[
            pl.BlockSpec((BR, D), lambda i: (i, 0)),
            pl.BlockSpec((D, 2), lambda i: (0, 0)),
            pl.BlockSpec((1, 2), lambda i: (0, 0)),
            pl.BlockSpec((1, 2), lambda i: (0, 0)),
        ],
        out_specs=[
            pl.BlockSpec((BR, 1), lambda i: (i, 0)),
            pl.BlockSpec((BR, 1), lambda i: (i, 0)),
        ],
        out_shape=[
            jax.ShapeDtypeStruct((V1, 1), jnp.float32),
            jax.ShapeDtypeStruct((V1, 1), jnp.float32),
        ],
    )


@functools.lru_cache(maxsize=None)
def _make_sc(B, L):
    RW = B // _NW
    CH = 64
    NCH = RW // CH
    IDXN = CH * L
    G16 = CH // 16
    U = 8
    assert L % U == 0 and RW % CH == 0 and RW % 16 == 0

    mesh = plsc.VectorSubcoreMesh(core_axis_name="c", subcore_axis_name="s")

    def body(seq_hbm, tgt_hbm, q1_hbm, p2_hbm, out_hbm,
             idx0, idx1, val0, val1, tgti, tgtv, outv, s0, s1, st):
        c = lax.axis_index("c")
        s = lax.axis_index("s")
        wid = s * _NC + c
        rbase = wid * RW
        fbase = rbase * L

        pltpu.sync_copy(tgt_hbm.at[pl.ds(rbase, RW)], tgti)
        tcp = pltpu.async_copy(p2_hbm.at[tgti], tgtv, st)

        idx = (idx0, idx1)
        val = (val0, val1)
        sem = (s0, s1)
        pltpu.sync_copy(seq_hbm.at[pl.ds(fbase, IDXN)], idx0)
        cps = [pltpu.async_copy(q1_hbm.at[idx0], val0, s0), None]

        iota16 = lax.broadcasted_iota(jnp.int32, (16,), 0)

        for g in range(NCH):
            cur, nxt = g % 2, (g + 1) % 2
            if g + 1 < NCH:
                pltpu.sync_copy(
                    seq_hbm.at[pl.ds(fbase + (g + 1) * IDXN, IDXN)], idx[nxt])
                cps[nxt] = pltpu.async_copy(q1_hbm.at[idx[nxt]], val[nxt], sem[nxt])
            cps[cur].wait()
            vref = val[cur]
            for grp in range(G16):
                iv0 = iota16 * L + (grp * 16 * L)

                def red_body(_, carry, vref=vref):
                    acc, iv = carry
                    for _u in range(U):
                        acc = acc + plsc.load_gather(vref, [iv])
                        iv = iv + 1
                    return acc, iv

                acc, _ = lax.fori_loop(
                    0, L // U, red_body,
                    (jnp.zeros((16,), jnp.float32), iv0))
                outv[pl.ds(g * CH + grp * 16, 16)] = acc

        tcp.wait()
        for i in range(RW // 16):
            sl = pl.ds(i * 16, 16)
            outv[sl] = outv[sl] + tgtv[sl]
        pltpu.sync_copy(outv, out_hbm.at[pl.ds(rbase, RW)])

    return pl.kernel(
        body,
        out_type=jax.ShapeDtypeStruct((B,), jnp.float32),
        mesh=mesh,
        compiler_params=pltpu.CompilerParams(needs_layout_passes=False),
        scratch_types=[
            pltpu.VMEM((IDXN,), jnp.int32),
            pltpu.VMEM((IDXN,), jnp.int32),
            pltpu.VMEM((IDXN,), jnp.float32),
            pltpu.VMEM((IDXN,), jnp.float32),
            pltpu.VMEM((RW,), jnp.int32),
            pltpu.VMEM((RW,), jnp.float32),
            pltpu.VMEM((RW,), jnp.float32),
            pltpu.SemaphoreType.DMA,
            pltpu.SemaphoreType.DMA,
            pltpu.SemaphoreType.DMA,
        ],
    )


def kernel(input_seq, target_item, table, W, b):
    B, L = input_seq.shape
    V1, D = table.shape
    Wt = jnp.transpose(W.reshape(2, D), (1, 0))
    scale = jnp.array([[1.0 / L, 1.0]], jnp.float32)
    bias = jnp.concatenate(
        [jnp.zeros((1, 1), jnp.float32), b.reshape(1, 1)], axis=1)
    q1, p2 = _make_proj(V1, D)(table, Wt, scale, bias)
    seq_flat = input_seq.astype(jnp.int32).reshape(B * L)
    out = _make_sc(B, L)(
        seq_flat, target_item.astype(jnp.int32),
        q1.reshape(V1), p2.reshape(V1))
    return out.reshape(B, 1)


# speedup vs baseline: 13.7750x; 1.6701x over previous
import functools

import jax
import jax.numpy as jnp
from jax import lax
from jax.experimental import pallas as pl
from jax.experimental.pallas import tpu as pltpu
from jax.experimental.pallas import tpu_sc as plsc

_NC = 2
_NS = 16
_NW = _NC * _NS


@functools.lru_cache(maxsize=None)
def _make_proj(V1, D, BR=2048):
    grid = (V1 + BR - 1) // BR

    def body(tab_ref, wt_ref, s_ref, bz_ref, q_ref):
        q = lax.dot_general(
            wt_ref[...], tab_ref[...],
            (((1,), (1,)), ((), ())),
            preferred_element_type=jnp.float32,
        )
        q_ref[...] = q * s_ref[...] + bz_ref[...]

    return pl.pallas_call(
        body,
        grid=(grid,),
        in_specs=[
            pl.BlockSpec((BR, D), lambda i: (i, 0)),
            pl.BlockSpec((2, D), lambda i: (0, 0)),
            pl.BlockSpec((2, 1), lambda i: (0, 0)),
            pl.BlockSpec((2, 1), lambda i: (0, 0)),
        ],
        out_specs=pl.BlockSpec((2, BR), lambda i: (0, i)),
        out_shape=jax.ShapeDtypeStruct((2, V1), jnp.float32),
    )


@functools.lru_cache(maxsize=None)
def _make_sc(B, L, V1):
    RW = B // _NW
    CH = 64
    NCH = RW // CH
    IDXN = CH * L
    G16 = CH // 16
    U = 8
    assert L % U == 0 and RW % CH == 0 and RW % 16 == 0

    mesh = plsc.VectorSubcoreMesh(core_axis_name="c", subcore_axis_name="s")

    def body(seq_hbm, tgt_hbm, qf_hbm, out_hbm,
             idx0, idx1, val0, val1, tgti, tgtv, outv, s0, s1, st):
        c = lax.axis_index("c")
        s = lax.axis_index("s")
        wid = s * _NC + c
        rbase = wid * RW
        fbase = rbase * L

        pltpu.sync_copy(tgt_hbm.at[pl.ds(rbase, RW)], tgti)
        for i in range(RW // 16):
            sl = pl.ds(i * 16, 16)
            tgti[sl] = tgti[sl] + V1
        tcp = pltpu.async_copy(qf_hbm.at[tgti], tgtv, st)

        idx = (idx0, idx1)
        val = (val0, val1)
        sem = (s0, s1)
        pltpu.sync_copy(seq_hbm.at[pl.ds(fbase, IDXN)], idx0)
        cps = [pltpu.async_copy(qf_hbm.at[idx0], val0, s0), None]

        iota16 = lax.broadcasted_iota(jnp.int32, (16,), 0)

        for g in range(NCH):
            cur, nxt = g % 2, (g + 1) % 2
            if g + 1 < NCH:
                pltpu.sync_copy(
                    seq_hbm.at[pl.ds(fbase + (g + 1) * IDXN, IDXN)], idx[nxt])
                cps[nxt] = pltpu.async_copy(qf_hbm.at[idx[nxt]], val[nxt], sem[nxt])
            cps[cur].wait()
            vref = val[cur]
            for grp in range(G16):
                iv0 = iota16 * L + (grp * 16 * L)

                def red_body(_, carry, vref=vref):
                    acc, iv = carry
                    for _u in range(U):
                        acc = acc + plsc.load_gather(vref, [iv])
                        iv = iv + 1
                    return acc, iv

                acc, _ = lax.fori_loop(
                    0, L // U, red_body,
                    (jnp.zeros((16,), jnp.float32), iv0))
                outv[pl.ds(g * CH + grp * 16, 16)] = acc

        tcp.wait()
        for i in range(RW // 16):
            sl = pl.ds(i * 16, 16)
            outv[sl] = outv[sl] + tgtv[sl]
        pltpu.sync_copy(outv, out_hbm.at[pl.ds(rbase, RW)])

    return pl.kernel(
        body,
        out_type=jax.ShapeDtypeStruct((B,), jnp.float32),
        mesh=mesh,
        compiler_params=pltpu.CompilerParams(needs_layout_passes=False),
        scratch_types=[
            pltpu.VMEM((IDXN,), jnp.int32),
            pltpu.VMEM((IDXN,), jnp.int32),
            pltpu.VMEM((IDXN,), jnp.float32),
            pltpu.VMEM((IDXN,), jnp.float32),
            pltpu.VMEM((RW,), jnp.int32),
            pltpu.VMEM((RW,), jnp.float32),
            pltpu.VMEM((RW,), jnp.float32),
            pltpu.SemaphoreType.DMA,
            pltpu.SemaphoreType.DMA,
            pltpu.SemaphoreType.DMA,
        ],
    )


def kernel(input_seq, target_item, table, W, b):
    B, L = input_seq.shape
    V1, D = table.shape
    scale = jnp.array([[1.0 / L], [1.0]], jnp.float32)
    bias = jnp.concatenate(
        [jnp.zeros((1, 1), jnp.float32), b.reshape(1, 1)], axis=0)
    q = _make_proj(V1, D)(table, W.reshape(2, D), scale, bias)
    seq_flat = input_seq.astype(jnp.int32).reshape(B * L)
    out = _make_sc(B, L, V1)(
        seq_flat, target_item.astype(jnp.int32), q.reshape(2 * V1))
    return out.reshape(B, 1)


# speedup vs baseline: 62.5789x; 4.5429x over previous
import functools

import jax
import jax.numpy as jnp
from jax import lax
from jax.experimental import pallas as pl
from jax.experimental.pallas import tpu as pltpu
from jax.experimental.pallas import tpu_sc as plsc

_NC = 2
_NS = 16
_NW = _NC * _NS


@functools.lru_cache(maxsize=None)
def _make_proj(V1, D, V1p, L, BR=8192):
    grid = (V1p + BR - 1) // BR
    inv_l = 1.0 / L

    def body(tab_ref, w_ref, b_ref, q1_ref, p2_ref):
        q = lax.dot_general(
            w_ref[...], tab_ref[...],
            (((1,), (0,)), ((), ())),
            preferred_element_type=jnp.float32,
        )
        q1_ref[...] = q[0, :] * inv_l
        p2_ref[...] = q[1, :] + b_ref[...][0, 0]

    return pl.pallas_call(
        body,
        grid=(grid,),
        in_specs=[
            pl.BlockSpec((D, BR), lambda i: (0, i)),
            pl.BlockSpec((2, D), lambda i: (0, 0)),
            pl.BlockSpec((1, 1), lambda i: (0, 0)),
        ],
        out_specs=[
            pl.BlockSpec((BR,), lambda i: (i,)),
            pl.BlockSpec((BR,), lambda i: (i,)),
        ],
        out_shape=[
            jax.ShapeDtypeStruct((V1p,), jnp.float32),
            jax.ShapeDtypeStruct((V1p,), jnp.float32),
        ],
    )


@functools.lru_cache(maxsize=None)
def _make_sc(B, L, V1p):
    RW = B // _NW
    CH = 64
    NCH = RW // CH
    IDXN = CH * L
    G16 = CH // 16
    U = 8
    assert L % U == 0 and RW % CH == 0 and RW % 16 == 0

    mesh = plsc.VectorSubcoreMesh(core_axis_name="c", subcore_axis_name="s")

    def body(seq_hbm, tgt_hbm, q1_hbm, p2_hbm, out_hbm,
             q1_sp, idx0, idx1, val0, val1, tgti, tgtv, outv, s0, s1, st):
        c = lax.axis_index("c")
        s = lax.axis_index("s")
        wid = s * _NC + c
        rbase = wid * RW
        fbase = rbase * L

        @pl.when(s == 0)
        def _stage():
            pltpu.sync_copy(q1_hbm, q1_sp)

        pltpu.sync_copy(tgt_hbm.at[pl.ds(rbase, RW)], tgti)
        tcp = pltpu.async_copy(p2_hbm.at[tgti], tgtv, st)

        idx = (idx0, idx1)
        val = (val0, val1)
        sem = (s0, s1)
        pltpu.sync_copy(seq_hbm.at[pl.ds(fbase, IDXN)], idx0)
        plsc.subcore_barrier()
        cps = [pltpu.async_copy(q1_sp.at[idx0], val0, s0), None]

        iota16 = lax.broadcasted_iota(jnp.int32, (16,), 0)

        for g in range(NCH):
            cur, nxt = g % 2, (g + 1) % 2
            if g + 1 < NCH:
                pltpu.sync_copy(
                    seq_hbm.at[pl.ds(fbase + (g + 1) * IDXN, IDXN)], idx[nxt])
                cps[nxt] = pltpu.async_copy(q1_sp.at[idx[nxt]], val[nxt], sem[nxt])
            cps[cur].wait()
            vref = val[cur]
            for grp in range(G16):
                iv0 = iota16 * L + (grp * 16 * L)

                def red_body(_, carry, vref=vref):
                    acc, iv = carry
                    for _u in range(U):
                        acc = acc + plsc.load_gather(vref, [iv])
                        iv = iv + 1
                    return acc, iv

                acc, _ = lax.fori_loop(
                    0, L // U, red_body,
                    (jnp.zeros((16,), jnp.float32), iv0))
                outv[pl.ds(g * CH + grp * 16, 16)] = acc

        tcp.wait()
        for i in range(RW // 16):
            sl = pl.ds(i * 16, 16)
            outv[sl] = outv[sl] + tgtv[sl]
        pltpu.sync_copy(outv, out_hbm.at[pl.ds(rbase, RW)])

    return pl.kernel(
        body,
        out_type=jax.ShapeDtypeStruct((B,), jnp.float32),
        mesh=mesh,
        compiler_params=pltpu.CompilerParams(needs_layout_passes=False),
        scratch_types=[
            pltpu.VMEM_SHARED((V1p,), jnp.float32),
            pltpu.VMEM((IDXN,), jnp.int32),
            pltpu.VMEM((IDXN,), jnp.int32),
            pltpu.VMEM((IDXN,), jnp.float32),
            pltpu.VMEM((IDXN,), jnp.float32),
            pltpu.VMEM((RW,), jnp.int32),
            pltpu.VMEM((RW,), jnp.float32),
            pltpu.VMEM((RW,), jnp.float32),
            pltpu.SemaphoreType.DMA,
            pltpu.SemaphoreType.DMA,
            pltpu.SemaphoreType.DMA,
        ],
    )


def kernel(input_seq, target_item, table, W, b):
    B, L = input_seq.shape
    V1, D = table.shape
    V1p = (V1 + 127) // 128 * 128
    q1, p2 = _make_proj(V1, D, V1p, L)(
        jnp.transpose(table), W.reshape(2, D), b.reshape(1, 1))
    seq_flat = input_seq.astype(jnp.int32).reshape(B * L)
    out = _make_sc(B, L, V1p)(
        seq_flat, target_item.astype(jnp.int32), q1, p2)
    return out.reshape(B, 1)


# speedup vs baseline: 74.8069x; 1.1954x over previous
import functools

import jax
import jax.numpy as jnp
from jax import lax
from jax.experimental import pallas as pl
from jax.experimental.pallas import tpu as pltpu
from jax.experimental.pallas import tpu_sc as plsc

_NC = 2
_NS = 16
_NW = _NC * _NS


@functools.lru_cache(maxsize=None)
def _make_proj(V1, D, V1p, L, BR=16384):
    grid = (V1p + BR - 1) // BR
    inv_l = 1.0 / L

    def body(tab_ref, w_ref, b_ref, q1_ref, p2_ref):
        q = lax.dot_general(
            w_ref[...], tab_ref[...],
            (((1,), (0,)), ((), ())),
            preferred_element_type=jnp.float32,
        )
        q1_ref[...] = q[0, :] * inv_l
        p2_ref[...] = q[1, :] + b_ref[...][0, 0]

    return pl.pallas_call(
        body,
        grid=(grid,),
        in_specs=[
            pl.BlockSpec((D, BR), lambda i: (0, i)),
            pl.BlockSpec((2, D), lambda i: (0, 0)),
            pl.BlockSpec((1, 1), lambda i: (0, 0)),
        ],
        out_specs=[
            pl.BlockSpec((BR,), lambda i: (i,)),
            pl.BlockSpec((BR,), lambda i: (i,)),
        ],
        out_shape=[
            jax.ShapeDtypeStruct((V1p,), jnp.float32),
            jax.ShapeDtypeStruct((V1p,), jnp.float32),
        ],
    )


@functools.lru_cache(maxsize=None)
def _make_sc(B, L, V1p):
    RW = B // _NW
    CH = 64
    NCH = RW // CH
    IDXN = CH * L
    G16 = CH // 16
    U = 8
    assert L % U == 0 and RW % CH == 0 and RW % 16 == 0

    mesh = plsc.VectorSubcoreMesh(core_axis_name="c", subcore_axis_name="s")

    def body(seq_hbm, tgt_hbm, q1_hbm, p2_hbm, out_hbm,
             q1_sp, idx0, idx1, val0, val1, tgti, tgtv, outv, s0, s1, st):
        c = lax.axis_index("c")
        s = lax.axis_index("s")
        wid = s * _NC + c
        rbase = wid * RW
        fbase = rbase * L

        @pl.when(s == 0)
        def _stage():
            pltpu.sync_copy(q1_hbm, q1_sp)

        pltpu.sync_copy(tgt_hbm.at[pl.ds(rbase, RW)], tgti)
        tcp = pltpu.async_copy(p2_hbm.at[tgti], tgtv, st)

        idx = (idx0, idx1)
        val = (val0, val1)
        sem = (s0, s1)
        pltpu.sync_copy(seq_hbm.at[pl.ds(fbase, IDXN)], idx0)
        plsc.subcore_barrier()
        cps = [pltpu.async_copy(q1_sp.at[idx0], val0, s0), None]

        iota16 = lax.broadcasted_iota(jnp.int32, (16,), 0)

        for g in range(NCH):
            cur, nxt = g % 2, (g + 1) % 2
            if g + 1 < NCH:
                pltpu.sync_copy(
                    seq_hbm.at[pl.ds(fbase + (g + 1) * IDXN, IDXN)], idx[nxt])
                cps[nxt] = pltpu.async_copy(q1_sp.at[idx[nxt]], val[nxt], sem[nxt])
            cps[cur].wait()
            vref = val[cur]
            for grp in range(G16):
                iv0 = iota16 * L + (grp * 16 * L)

                def red_body(_, carry, vref=vref):
                    acc, iv = carry
                    for _u in range(U):
                        acc = acc + plsc.load_gather(vref, [iv])
                        iv = iv + 1
                    return acc, iv

                acc, _ = lax.fori_loop(
                    0, L // U, red_body,
                    (jnp.zeros((16,), jnp.float32), iv0))
                outv[pl.ds(g * CH + grp * 16, 16)] = acc

        tcp.wait()
        for i in range(RW // 16):
            sl = pl.ds(i * 16, 16)
            outv[sl] = outv[sl] + tgtv[sl]
        pltpu.sync_copy(outv, out_hbm.at[pl.ds(rbase, RW)])

    return pl.kernel(
        body,
        out_type=jax.ShapeDtypeStruct((B,), jnp.float32),
        mesh=mesh,
        compiler_params=pltpu.CompilerParams(needs_layout_passes=False),
        scratch_types=[
            pltpu.VMEM_SHARED((V1p,), jnp.float32),
            pltpu.VMEM((IDXN,), jnp.int32),
            pltpu.VMEM((IDXN,), jnp.int32),
            pltpu.VMEM((IDXN,), jnp.float32),
            pltpu.VMEM((IDXN,), jnp.float32),
            pltpu.VMEM((RW,), jnp.int32),
            pltpu.VMEM((RW,), jnp.float32),
            pltpu.VMEM((RW,), jnp.float32),
            pltpu.SemaphoreType.DMA,
            pltpu.SemaphoreType.DMA,
            pltpu.SemaphoreType.DMA,
        ],
    )


def kernel(input_seq, target_item, table, W, b):
    B, L = input_seq.shape
    V1, D = table.shape
    V1p = (V1 + 127) // 128 * 128
    q1, p2 = _make_proj(V1, D, V1p, L)(
        jnp.transpose(table), W.reshape(2, D), b.reshape(1, 1))
    seq_flat = input_seq.astype(jnp.int32).reshape(B * L)
    out = _make_sc(B, L, V1p)(
        seq_flat, target_item.astype(jnp.int32), q1, p2)
    return out.reshape(B, 1)
